# Initial kernel scaffold; baseline (speedup 1.0000x reference)
import functools

import jax
import jax.numpy as jnp
from jax import lax
from jax.experimental import pallas as pl
from jax.experimental.pallas import tpu as pltpu
from jax.experimental.pallas import tpu_sc as plsc

F = 26
V = 100000
E = 16
NUMERIC = 13
B = 16384
H1, H2, H3 = 256, 128, 3
EPS = 1e-5

NC = 2
NS = 16
NW = NC * NS

ROWS = B * F
RPW = ROWS // NW
CH = 128
NCH = RPW // CH
NBUF = 8
NG = NCH // NBUF


def _sc_gather_body(tab_hbm, idx_hbm, out_hbm, idx_v, rows_v, gsem, wsem):
    c = lax.axis_index("c")
    s = lax.axis_index("s")
    wid = s * NC + c
    base = wid * RPW
    pltpu.sync_copy(idx_hbm.at[wid], idx_v)

    def group(g, carry):
        @pl.when(g > 0)
        def _():
            for b in range(NBUF):
                pltpu.make_async_copy(
                    rows_v.at[b], out_hbm.at[pl.ds(base, CH)], wsem
                ).wait()

        gathers = []
        for b in range(NBUF):
            j = g * NBUF + b
            d = pltpu.make_async_copy(
                tab_hbm.at[idx_v.at[j]], rows_v.at[b], gsem
            )
            d.start()
            gathers.append(d)
        for d in gathers:
            d.wait()
        for b in range(NBUF):
            j = g * NBUF + b
            pltpu.make_async_copy(
                rows_v.at[b], out_hbm.at[pl.ds(base + j * CH, CH)], wsem
            ).start()
        return carry

    lax.fori_loop(0, NG, group, 0)
    for b in range(NBUF):
        pltpu.make_async_copy(
            rows_v.at[b], out_hbm.at[pl.ds(base, CH)], wsem
        ).wait()


_sc_gather = functools.partial(
    pl.kernel,
    out_type=jax.ShapeDtypeStruct((ROWS, E), jnp.float32),
    mesh=plsc.VectorSubcoreMesh(core_axis_name="c", subcore_axis_name="s"),
    scratch_types=[
        pltpu.VMEM((NCH, CH), jnp.int32),
        pltpu.VMEM((NBUF, CH, E), jnp.float32),
        pltpu.SemaphoreType.DMA,
        pltpu.SemaphoreType.DMA,
    ],
)(_sc_gather_body)


BT = 2048


def _mlp_body(xn_ref, emb_ref, w1n_ref, w1e_ref, b1_ref, g1_ref, be1_ref,
              w2_ref, b2_ref, g2_ref, be2_ref, w3_ref, b3_ref, out_ref):
    s = 1.0 / jnp.sqrt(1.0 + EPS)
    h = jnp.dot(xn_ref[...], w1n_ref[...], preferred_element_type=jnp.float32)
    h = h + jnp.dot(emb_ref[...], w1e_ref[...],
                    preferred_element_type=jnp.float32)
    h = (h + b1_ref[...]) * (g1_ref[...] * s) + be1_ref[...]
    h = jnp.maximum(h, 0.0)
    h = jnp.dot(h, w2_ref[...], preferred_element_type=jnp.float32)
    h = (h + b2_ref[...]) * (g2_ref[...] * s) + be2_ref[...]
    h = jnp.maximum(h, 0.0)
    out_ref[...] = (
        jnp.dot(h, w3_ref[...], preferred_element_type=jnp.float32)
        + b3_ref[...]
    )


def _mlp(xn, emb, w1n, w1e, b1, g1, be1, w2, b2, g2, be2, w3, b3):
    full = lambda shape: pl.BlockSpec(shape, lambda i: (0,) * len(shape))
    return pl.pallas_call(
        _mlp_body,
        grid=(B // BT,),
        in_specs=[
            pl.BlockSpec((BT, NUMERIC), lambda i: (i, 0)),
            pl.BlockSpec((BT, F * E), lambda i: (i, 0)),
            full((NUMERIC, H1)),
            full((F * E, H1)),
            full((1, H1)),
            full((1, H1)),
            full((1, H1)),
            full((H1, H2)),
            full((1, H2)),
            full((1, H2)),
            full((1, H2)),
            full((H2, H3)),
            full((1, H3)),
        ],
        out_specs=pl.BlockSpec((BT, H3), lambda i: (i, 0)),
        out_shape=jax.ShapeDtypeStruct((B, H3), jnp.float32),
    )(xn, emb, w1n, w1e, b1, g1, be1, w2, b2, g2, be2, w3, b3)


def kernel(x_numeric, x_cat, tables, W1, b1, g1, be1, W2, b2, g2, be2, W3, b3):
    flat_tables = tables.reshape(F * V, E)
    offs = (jnp.arange(F, dtype=jnp.int32) * V)[None, :]
    idx = (jnp.clip(x_cat, 0, V - 1) + offs).reshape(NW, NCH, CH)
    emb = _sc_gather(flat_tables, idx).reshape(B, F * E)
    return _mlp(
        x_numeric, emb, W1[:NUMERIC], W1[NUMERIC:],
        b1[None, :], g1[None, :], be1[None, :],
        W2, b2[None, :], g2[None, :], be2[None, :],
        W3, b3[None, :],
    )


# speedup vs baseline: 7.7780x; 7.7780x over previous
import functools

import jax
import jax.numpy as jnp
from jax import lax
from jax.experimental import pallas as pl
from jax.experimental.pallas import tpu as pltpu
from jax.experimental.pallas import tpu_sc as plsc

F = 26
V = 100000
E = 16
NUMERIC = 13
B = 16384
H1, H2, H3 = 256, 128, 3
EPS = 1e-5

NC = 2
NS = 16
NW = NC * NS

ROWS = B * F
RPW = ROWS // NW
CH = 128
NCH = RPW // CH
NBUF = 8
NG = NCH // NBUF


def _sc_gather_body(tab_hbm, idx_hbm, out_hbm, idx_v, rows_v, gsem, wsem):
    c = lax.axis_index("c")
    s = lax.axis_index("s")
    wid = s * NC + c
    base = wid * RPW
    pltpu.sync_copy(idx_hbm.at[wid], idx_v)

    def group(g, carry):
        @pl.when(g > 0)
        def _():
            for b in range(NBUF):
                pltpu.make_async_copy(
                    rows_v.at[b], out_hbm.at[pl.ds(base, CH)], wsem
                ).wait()

        gathers = []
        for b in range(NBUF):
            j = g * NBUF + b
            d = pltpu.make_async_copy(
                tab_hbm.at[idx_v.at[j]], rows_v.at[b], gsem
            )
            d.start()
            gathers.append(d)
        for d in gathers:
            d.wait()
        for b in range(NBUF):
            j = g * NBUF + b
            pltpu.make_async_copy(
                rows_v.at[b], out_hbm.at[pl.ds(base + j * CH, CH)], wsem
            ).start()
        return carry

    lax.fori_loop(0, NG, group, 0)
    for b in range(NBUF):
        pltpu.make_async_copy(
            rows_v.at[b], out_hbm.at[pl.ds(base, CH)], wsem
        ).wait()


@functools.cache
def _sc_gather():
    return functools.partial(
        pl.kernel,
        out_type=jax.ShapeDtypeStruct((ROWS, E), jnp.float32),
        mesh=plsc.VectorSubcoreMesh(core_axis_name="c", subcore_axis_name="s"),
        scratch_types=[
            pltpu.VMEM((NCH, CH), jnp.int32),
            pltpu.VMEM((NBUF, CH, E), jnp.float32),
            pltpu.SemaphoreType.DMA,
            pltpu.SemaphoreType.DMA,
        ],
        compiler_params=pltpu.CompilerParams(use_tc_tiling_on_sc=False),
    )(_sc_gather_body)


BT = 2048


def _mlp_body(xn_ref, emb_ref, w1n_ref, w1e_ref, b1_ref, g1_ref, be1_ref,
              w2_ref, b2_ref, g2_ref, be2_ref, w3_ref, b3_ref, out_ref):
    s = 1.0 / jnp.sqrt(1.0 + EPS)
    h = jnp.dot(xn_ref[...], w1n_ref[...], preferred_element_type=jnp.float32)
    h = h + jnp.dot(emb_ref[...], w1e_ref[...],
                    preferred_element_type=jnp.float32)
    h = (h + b1_ref[...]) * (g1_ref[...] * s) + be1_ref[...]
    h = jnp.maximum(h, 0.0)
    h = jnp.dot(h, w2_ref[...], preferred_element_type=jnp.float32)
    h = (h + b2_ref[...]) * (g2_ref[...] * s) + be2_ref[...]
    h = jnp.maximum(h, 0.0)
    out_ref[...] = (
        jnp.dot(h, w3_ref[...], preferred_element_type=jnp.float32)
        + b3_ref[...]
    )


def _mlp(xn, emb, w1n, w1e, b1, g1, be1, w2, b2, g2, be2, w3, b3):
    full = lambda shape: pl.BlockSpec(shape, lambda i: (0,) * len(shape))
    return pl.pallas_call(
        _mlp_body,
        grid=(B // BT,),
        in_specs=[
            pl.BlockSpec((BT, NUMERIC), lambda i: (i, 0)),
            pl.BlockSpec((BT, F * E), lambda i: (i, 0)),
            full((NUMERIC, H1)),
            full((F * E, H1)),
            full((1, H1)),
            full((1, H1)),
            full((1, H1)),
            full((H1, H2)),
            full((1, H2)),
            full((1, H2)),
            full((1, H2)),
            full((H2, H3)),
            full((1, H3)),
        ],
        out_specs=pl.BlockSpec((BT, H3), lambda i: (i, 0)),
        out_shape=jax.ShapeDtypeStruct((B, H3), jnp.float32),
    )(xn, emb, w1n, w1e, b1, g1, be1, w2, b2, g2, be2, w3, b3)


def kernel(x_numeric, x_cat, tables, W1, b1, g1, be1, W2, b2, g2, be2, W3, b3):
    flat_tables = tables.reshape(F * V, E)
    offs = (jnp.arange(F, dtype=jnp.int32) * V)[None, :]
    idx = (jnp.clip(x_cat, 0, V - 1) + offs).reshape(NW, NCH, CH)
    emb = _sc_gather()(flat_tables, idx).reshape(B, F * E)
    return _mlp(
        x_numeric, emb, W1[:NUMERIC], W1[NUMERIC:],
        b1[None, :], g1[None, :], be1[None, :],
        W2, b2[None, :], g2[None, :], be2[None, :],
        W3, b3[None, :],
    )


# speedup vs baseline: 46.2858x; 5.9509x over previous
import functools

import jax
import jax.numpy as jnp
from jax import lax
from jax.experimental import pallas as pl
from jax.experimental.pallas import tpu as pltpu
from jax.experimental.pallas import tpu_sc as plsc

F = 26
V = 100000
E = 16
NUMERIC = 13
B = 16384
H1, H2, H3 = 256, 128, 3
EPS = 1e-5

NC = 2
NS = 16
NW = NC * NS

TASKS = F * E
TPW = TASKS // NW
NPASS = 4
QV = 25088
TAIL = V - 74880 - QV
PASSES = (
    (0, 0, 0, QV, QV),
    (QV, QV, QV, 2 * QV, QV),
    (2 * QV, 0, 2 * QV, 3 * QV, QV),
    (74880, QV, 3 * QV, V, QV + TAIL),
)


def _sc_gather_body(tab_hbm, tail_hbm, idx_hbm, out_hbm, row_v, idx_v, out_v,
                    sem, tsem, isem, osem0, osem1):
    c = lax.axis_index("c")
    s = lax.axis_index("s")
    wid = s * NC + c
    tid0 = wid * TPW

    def rowcp(t, q):
        tid = tid0 + t
        src, slot, _, _, _ = PASSES[q]
        return pltpu.make_async_copy(
            tab_hbm.at[tid // E, tid % E, pl.ds(src, QV)],
            row_v.at[pl.ds(slot, QV)],
            sem,
        )

    def tailcp(t):
        tid = tid0 + t
        return pltpu.make_async_copy(
            tail_hbm.at[tid // E, tid % E],
            row_v.at[pl.ds(2 * QV, 128)],
            tsem,
        )

    def idxcp(t):
        tid = tid0 + t
        return pltpu.make_async_copy(
            idx_hbm.at[tid // E],
            idx_v.at[pl.ds(lax.rem(t, 2) * B, B)],
            isem,
        )

    def outcp(t, sem_):
        return pltpu.make_async_copy(
            out_v.at[pl.ds(lax.rem(t, 2) * B, B)], out_hbm.at[tid0 + t], sem_
        )

    rowcp(0, 0).start()
    idxcp(0).start()

    def task(t, carry):
        ob = lax.rem(t, 2)
        obase = ob * B
        idxcp(t).wait()

        @pl.when(t + 1 < TPW)
        def _():
            idxcp(t + 1).start()

        @pl.when((t >= 2) & (ob == 0))
        def _():
            outcp(t, osem0).wait()

        @pl.when((t >= 2) & (ob == 1))
        def _():
            outcp(t, osem1).wait()

        for q in range(NPASS):
            rowcp(t, q).wait()
            if q + 1 < NPASS:
                rowcp(t, q + 1).start()
                if q + 1 == NPASS - 1:
                    tailcp(t).start()
            else:
                tailcp(t).wait()

                @pl.when(t + 1 < TPW)
                def _():
                    rowcp(t + 1, 0).start()

            lo, slot, mlo, mhi, ln = PASSES[q]

            @plsc.parallel_loop(0, B // 16, 1, unroll=8)
            def g16(k):
                i = obase + k * 16
                idx16 = idx_v[pl.ds(i, 16)]
                loc = jnp.clip(idx16 - lo, 0, ln - 1) + slot
                v = plsc.load_gather(row_v, [loc])
                m = (idx16 >= mlo) & (idx16 < mhi)
                pos = i + lax.iota(jnp.int32, 16)
                plsc.store_scatter(out_v, [pos], v, mask=m)

        @pl.when(ob == 0)
        def _():
            outcp(t, osem0).start()

        @pl.when(ob == 1)
        def _():
            outcp(t, osem1).start()

        return carry

    lax.fori_loop(0, TPW, task, 0)
    outcp(0, osem0).wait()
    outcp(0, osem1).wait()


@functools.cache
def _sc_gather():
    return functools.partial(
        pl.kernel,
        out_type=jax.ShapeDtypeStruct((TASKS, B), jnp.float32),
        mesh=plsc.VectorSubcoreMesh(core_axis_name="c", subcore_axis_name="s"),
        scratch_types=[
            pltpu.VMEM((2 * QV + 128,), jnp.float32),
            pltpu.VMEM((2 * B,), jnp.int32),
            pltpu.VMEM((2 * B,), jnp.float32),
            pltpu.SemaphoreType.DMA,
            pltpu.SemaphoreType.DMA,
            pltpu.SemaphoreType.DMA,
            pltpu.SemaphoreType.DMA,
            pltpu.SemaphoreType.DMA,
        ],
        compiler_params=pltpu.CompilerParams(
            use_tc_tiling_on_sc=True, needs_layout_passes=False
        ),
    )(_sc_gather_body)


BT = 2048


def _mlp_body(xn_ref, embt_ref, w1n_ref, w1e_ref, b1_ref, g1_ref, be1_ref,
              w2_ref, b2_ref, g2_ref, be2_ref, w3_ref, b3_ref, out_ref):
    s = 1.0 / jnp.sqrt(1.0 + EPS)
    h = jnp.dot(xn_ref[...], w1n_ref[...], preferred_element_type=jnp.float32)
    h = h + lax.dot_general(
        embt_ref[...], w1e_ref[...], (((0,), (0,)), ((), ())),
        preferred_element_type=jnp.float32,
    )
    h = (h + b1_ref[...]) * (g1_ref[...] * s) + be1_ref[...]
    h = jnp.maximum(h, 0.0)
    h = jnp.dot(h, w2_ref[...], preferred_element_type=jnp.float32)
    h = (h + b2_ref[...]) * (g2_ref[...] * s) + be2_ref[...]
    h = jnp.maximum(h, 0.0)
    out_ref[...] = (
        jnp.dot(h, w3_ref[...], preferred_element_type=jnp.float32)
        + b3_ref[...]
    )


def _mlp(xn, embt, w1n, w1e, b1, g1, be1, w2, b2, g2, be2, w3, b3):
    full = lambda shape: pl.BlockSpec(shape, lambda i: (0,) * len(shape))
    return pl.pallas_call(
        _mlp_body,
        grid=(B // BT,),
        in_specs=[
            pl.BlockSpec((BT, NUMERIC), lambda i: (i, 0)),
            pl.BlockSpec((TASKS, BT), lambda i: (0, i)),
            full((NUMERIC, H1)),
            full((TASKS, H1)),
            full((1, H1)),
            full((1, H1)),
            full((1, H1)),
            full((H1, H2)),
            full((1, H2)),
            full((1, H2)),
            full((1, H2)),
            full((H2, H3)),
            full((1, H3)),
        ],
        out_specs=pl.BlockSpec((BT, H3), lambda i: (i, 0)),
        out_shape=jax.ShapeDtypeStruct((B, H3), jnp.float32),
    )(xn, embt, w1n, w1e, b1, g1, be1, w2, b2, g2, be2, w3, b3)


def kernel(x_numeric, x_cat, tables, W1, b1, g1, be1, W2, b2, g2, be2, W3, b3):
    tabT = jnp.transpose(tables, (0, 2, 1))
    tail = jnp.pad(tabT[:, :, V - TAIL:], ((0, 0), (0, 0), (0, 128 - TAIL)))
    idxT = jnp.clip(x_cat, 0, V - 1).T.astype(jnp.int32)
    embT = _sc_gather()(tabT, tail, idxT)
    return _mlp(
        x_numeric, embT, W1[:NUMERIC], W1[NUMERIC:],
        b1[None, :], g1[None, :], be1[None, :],
        W2, b2[None, :], g2[None, :], be2[None, :],
        W3, b3[None, :],
    )


# speedup vs baseline: 46.5375x; 1.0054x over previous
import functools

import jax
import jax.numpy as jnp
from jax import lax
from jax.experimental import pallas as pl
from jax.experimental.pallas import tpu as pltpu
from jax.experimental.pallas import tpu_sc as plsc

F = 26
V = 100000
E = 16
NUMERIC = 13
B = 16384
H1, H2, H3 = 256, 128, 3
EPS = 1e-5

NC = 2
NS = 16
NW = NC * NS

TASKS = F * E
TPW = TASKS // NW
NPASS = 4
QV = 25088
TAIL = V - 74880 - QV
PASSES = (
    (0, 0, 0, QV, QV),
    (QV, QV, QV, 2 * QV, QV),
    (2 * QV, 0, 2 * QV, 3 * QV, QV),
    (74880, QV, 3 * QV, V, QV + TAIL),
)


def _sc_gather_body(tab_hbm, tail_hbm, idx_hbm, out_hbm, row_v, idx_v, out_v,
                    sem, tsem, isem, osem0, osem1):
    c = lax.axis_index("c")
    s = lax.axis_index("s")
    wid = s * NC + c
    tid0 = wid * TPW

    def rowcp(t, q):
        tid = tid0 + t
        src, slot, _, _, _ = PASSES[q]
        return pltpu.make_async_copy(
            tab_hbm.at[tid // E, tid % E, pl.ds(src, QV)],
            row_v.at[pl.ds(slot, QV)],
            sem,
        )

    def tailcp(t):
        tid = tid0 + t
        return pltpu.make_async_copy(
            tail_hbm.at[tid // E, tid % E],
            row_v.at[pl.ds(2 * QV, 128)],
            tsem,
        )

    def idxcp(t):
        tid = tid0 + t
        return pltpu.make_async_copy(
            idx_hbm.at[tid // E],
            idx_v.at[pl.ds(lax.rem(t, 2) * B, B)],
            isem,
        )

    def outcp(t, sem_):
        return pltpu.make_async_copy(
            out_v.at[pl.ds(lax.rem(t, 2) * B, B)], out_hbm.at[tid0 + t], sem_
        )

    rowcp(0, 0).start()
    idxcp(0).start()

    def task(t, carry):
        ob = lax.rem(t, 2)
        obase = ob * B
        idxcp(t).wait()

        @pl.when(t + 1 < TPW)
        def _():
            idxcp(t + 1).start()

        @pl.when((t >= 2) & (ob == 0))
        def _():
            outcp(t, osem0).wait()

        @pl.when((t >= 2) & (ob == 1))
        def _():
            outcp(t, osem1).wait()

        for q in range(NPASS):
            rowcp(t, q).wait()
            if q + 1 < NPASS:
                rowcp(t, q + 1).start()
                if q + 1 == NPASS - 1:
                    tailcp(t).start()
            else:
                tailcp(t).wait()

                @pl.when(t + 1 < TPW)
                def _():
                    rowcp(t + 1, 0).start()

            lo, slot, mlo, mhi, ln = PASSES[q]

            @plsc.parallel_loop(0, B // 16, 1, unroll=8)
            def g16(k):
                i = obase + k * 16
                idx16 = idx_v[pl.ds(i, 16)]
                loc = jnp.clip(idx16 - lo, 0, ln - 1) + slot
                v = plsc.load_gather(row_v, [loc])
                m = (idx16 >= mlo) & (idx16 < mhi)
                pos = i + lax.iota(jnp.int32, 16)
                plsc.store_scatter(out_v, [pos], v, mask=m)

        @pl.when(ob == 0)
        def _():
            outcp(t, osem0).start()

        @pl.when(ob == 1)
        def _():
            outcp(t, osem1).start()

        return carry

    lax.fori_loop(0, TPW, task, 0)
    outcp(0, osem0).wait()
    outcp(0, osem1).wait()


@functools.cache
def _sc_gather():
    return functools.partial(
        pl.kernel,
        out_type=jax.ShapeDtypeStruct((TASKS, B), jnp.float32),
        mesh=plsc.VectorSubcoreMesh(core_axis_name="c", subcore_axis_name="s"),
        scratch_types=[
            pltpu.VMEM((2 * QV + 128,), jnp.float32),
            pltpu.VMEM((2 * B,), jnp.int32),
            pltpu.VMEM((2 * B,), jnp.float32),
            pltpu.SemaphoreType.DMA,
            pltpu.SemaphoreType.DMA,
            pltpu.SemaphoreType.DMA,
            pltpu.SemaphoreType.DMA,
            pltpu.SemaphoreType.DMA,
        ],
        compiler_params=pltpu.CompilerParams(
            use_tc_tiling_on_sc=True, needs_layout_passes=False
        ),
    )(_sc_gather_body)


BT = 4096


def _mlp_body(xn_ref, embt_ref, w1n_ref, w1e_ref, b1_ref, g1_ref, be1_ref,
              w2_ref, b2_ref, g2_ref, be2_ref, w3_ref, b3_ref, out_ref):
    s = 1.0 / jnp.sqrt(1.0 + EPS)
    h = jnp.dot(xn_ref[...], w1n_ref[...], preferred_element_type=jnp.float32)
    h = h + lax.dot_general(
        embt_ref[...], w1e_ref[...], (((0,), (0,)), ((), ())),
        preferred_element_type=jnp.float32,
    )
    h = (h + b1_ref[...]) * (g1_ref[...] * s) + be1_ref[...]
    h = jnp.maximum(h, 0.0)
    h = jnp.dot(h, w2_ref[...], preferred_element_type=jnp.float32)
    h = (h + b2_ref[...]) * (g2_ref[...] * s) + be2_ref[...]
    h = jnp.maximum(h, 0.0)
    out_ref[...] = (
        jnp.dot(h, w3_ref[...], preferred_element_type=jnp.float32)
        + b3_ref[...]
    )


def _mlp(xn, embt, w1n, w1e, b1, g1, be1, w2, b2, g2, be2, w3, b3):
    full = lambda shape: pl.BlockSpec(shape, lambda i: (0,) * len(shape))
    return pl.pallas_call(
        _mlp_body,
        grid=(B // BT,),
        in_specs=[
            pl.BlockSpec((BT, NUMERIC), lambda i: (i, 0)),
            pl.BlockSpec((TASKS, BT), lambda i: (0, i)),
            full((NUMERIC, H1)),
            full((TASKS, H1)),
            full((1, H1)),
            full((1, H1)),
            full((1, H1)),
            full((H1, H2)),
            full((1, H2)),
            full((1, H2)),
            full((1, H2)),
            full((H2, H3)),
            full((1, H3)),
        ],
        out_specs=pl.BlockSpec((BT, H3), lambda i: (i, 0)),
        out_shape=jax.ShapeDtypeStruct((B, H3), jnp.float32),
    )(xn, embt, w1n, w1e, b1, g1, be1, w2, b2, g2, be2, w3, b3)


def kernel(x_numeric, x_cat, tables, W1, b1, g1, be1, W2, b2, g2, be2, W3, b3):
    tabT = jnp.transpose(tables, (0, 2, 1))
    tail = jnp.pad(tabT[:, :, V - TAIL:], ((0, 0), (0, 0), (0, 128 - TAIL)))
    idxT = jnp.clip(x_cat, 0, V - 1).T.astype(jnp.int32)
    embT = _sc_gather()(tabT, tail, idxT)
    return _mlp(
        x_numeric, embT, W1[:NUMERIC], W1[NUMERIC:],
        b1[None, :], g1[None, :], be1[None, :],
        W2, b2[None, :], g2[None, :], be2[None, :],
        W3, b3[None, :],
    )


# speedup vs baseline: 46.6430x; 1.0023x over previous
import functools

import jax
import jax.numpy as jnp
from jax import lax
from jax.experimental import pallas as pl
from jax.experimental.pallas import tpu as pltpu
from jax.experimental.pallas import tpu_sc as plsc

F = 26
V = 100000
E = 16
NUMERIC = 13
B = 16384
H1, H2, H3 = 256, 128, 3
EPS = 1e-5

NC = 2
NS = 16
NW = NC * NS

TASKS = F * E
TPW = TASKS // NW
NPASS = 4
QV = 25088
TAIL = V - 74880 - QV
PASSES = (
    (0, 0, 0, QV, QV),
    (QV, QV, QV, 2 * QV, QV),
    (2 * QV, 0, 2 * QV, 3 * QV, QV),
    (74880, QV, 3 * QV, V, QV + TAIL),
)


def _sc_gather_body(tab_hbm, tail_hbm, idx_hbm, out_hbm, row_v, idx_v, out_v,
                    sem0, sem1, tsem, isem, osem0, osem1):
    c = lax.axis_index("c")
    s = lax.axis_index("s")
    wid = s * NC + c
    tid0 = wid * TPW

    def rowcp(t, q):
        tid = tid0 + t
        src, slot, _, _, _ = PASSES[q]
        return pltpu.make_async_copy(
            tab_hbm.at[tid // E, tid % E, pl.ds(src, QV)],
            row_v.at[pl.ds(slot, QV)],
            sem0 if q % 2 == 0 else sem1,
        )

    def tailcp(t):
        tid = tid0 + t
        return pltpu.make_async_copy(
            tail_hbm.at[tid // E, tid % E],
            row_v.at[pl.ds(2 * QV, 128)],
            tsem,
        )

    def idxcp(t):
        tid = tid0 + t
        return pltpu.make_async_copy(
            idx_hbm.at[tid // E],
            idx_v.at[pl.ds(lax.rem(t, 2) * B, B)],
            isem,
        )

    def outcp(t, sem_):
        return pltpu.make_async_copy(
            out_v.at[pl.ds(lax.rem(t, 2) * B, B)], out_hbm.at[tid0 + t], sem_
        )

    rowcp(0, 0).start()
    idxcp(0).start()

    def task(t, carry):
        ob = lax.rem(t, 2)
        obase = ob * B
        idxcp(t).wait()

        @pl.when(t + 1 < TPW)
        def _():
            idxcp(t + 1).start()

        @pl.when((t >= 2) & (ob == 0))
        def _():
            outcp(t, osem0).wait()

        @pl.when((t >= 2) & (ob == 1))
        def _():
            outcp(t, osem1).wait()

        for q in range(NPASS):
            if q + 1 < NPASS:
                rowcp(t, q + 1).start()
                if q + 1 == NPASS - 1:
                    tailcp(t).start()
            else:
                @pl.when(t + 1 < TPW)
                def _():
                    rowcp(t + 1, 0).start()

            rowcp(t, q).wait()
            if q == NPASS - 1:
                tailcp(t).wait()

            lo, slot, mlo, mhi, ln = PASSES[q]

            @plsc.parallel_loop(0, B // 16, 1, unroll=8)
            def g16(k):
                i = obase + k * 16
                idx16 = idx_v[pl.ds(i, 16)]
                loc = jnp.clip(idx16 - lo, 0, ln - 1) + slot
                v = plsc.load_gather(row_v, [loc])
                m = (idx16 >= mlo) & (idx16 < mhi)
                pos = i + lax.iota(jnp.int32, 16)
                plsc.store_scatter(out_v, [pos], v, mask=m)

        @pl.when(ob == 0)
        def _():
            outcp(t, osem0).start()

        @pl.when(ob == 1)
        def _():
            outcp(t, osem1).start()

        return carry

    lax.fori_loop(0, TPW, task, 0)
    outcp(0, osem0).wait()
    outcp(0, osem1).wait()


@functools.cache
def _sc_gather():
    return functools.partial(
        pl.kernel,
        out_type=jax.ShapeDtypeStruct((TASKS, B), jnp.float32),
        mesh=plsc.VectorSubcoreMesh(core_axis_name="c", subcore_axis_name="s"),
        scratch_types=[
            pltpu.VMEM((2 * QV + 128,), jnp.float32),
            pltpu.VMEM((2 * B,), jnp.int32),
            pltpu.VMEM((2 * B,), jnp.float32),
            pltpu.SemaphoreType.DMA,
            pltpu.SemaphoreType.DMA,
            pltpu.SemaphoreType.DMA,
            pltpu.SemaphoreType.DMA,
            pltpu.SemaphoreType.DMA,
            pltpu.SemaphoreType.DMA,
        ],
        compiler_params=pltpu.CompilerParams(
            use_tc_tiling_on_sc=True, needs_layout_passes=False
        ),
    )(_sc_gather_body)


BT = 4096


def _mlp_body(xn_ref, embt_ref, w1n_ref, w1e_ref, b1_ref, g1_ref, be1_ref,
              w2_ref, b2_ref, g2_ref, be2_ref, w3_ref, b3_ref, out_ref):
    s = 1.0 / jnp.sqrt(1.0 + EPS)
    h = jnp.dot(xn_ref[...], w1n_ref[...], preferred_element_type=jnp.float32)
    h = h + lax.dot_general(
        embt_ref[...], w1e_ref[...], (((0,), (0,)), ((), ())),
        preferred_element_type=jnp.float32,
    )
    h = (h + b1_ref[...]) * (g1_ref[...] * s) + be1_ref[...]
    h = jnp.maximum(h, 0.0)
    h = jnp.dot(h, w2_ref[...], preferred_element_type=jnp.float32)
    h = (h + b2_ref[...]) * (g2_ref[...] * s) + be2_ref[...]
    h = jnp.maximum(h, 0.0)
    out_ref[...] = (
        jnp.dot(h, w3_ref[...], preferred_element_type=jnp.float32)
        + b3_ref[...]
    )


def _mlp(xn, embt, w1n, w1e, b1, g1, be1, w2, b2, g2, be2, w3, b3):
    full = lambda shape: pl.BlockSpec(shape, lambda i: (0,) * len(shape))
    return pl.pallas_call(
        _mlp_body,
        grid=(B // BT,),
        in_specs=[
            pl.BlockSpec((BT, NUMERIC), lambda i: (i, 0)),
            pl.BlockSpec((TASKS, BT), lambda i: (0, i)),
            full((NUMERIC, H1)),
            full((TASKS, H1)),
            full((1, H1)),
            full((1, H1)),
            full((1, H1)),
            full((H1, H2)),
            full((1, H2)),
            full((1, H2)),
            full((1, H2)),
            full((H2, H3)),
            full((1, H3)),
        ],
        out_specs=pl.BlockSpec((BT, H3), lambda i: (i, 0)),
        out_shape=jax.ShapeDtypeStruct((B, H3), jnp.float32),
    )(xn, embt, w1n, w1e, b1, g1, be1, w2, b2, g2, be2, w3, b3)


def kernel(x_numeric, x_cat, tables, W1, b1, g1, be1, W2, b2, g2, be2, W3, b3):
    tabT = jnp.transpose(tables, (0, 2, 1))
    tail = jnp.pad(tabT[:, :, V - TAIL:], ((0, 0), (0, 0), (0, 128 - TAIL)))
    idxT = jnp.clip(x_cat, 0, V - 1).T.astype(jnp.int32)
    embT = _sc_gather()(tabT, tail, idxT)
    return _mlp(
        x_numeric, embT, W1[:NUMERIC], W1[NUMERIC:],
        b1[None, :], g1[None, :], be1[None, :],
        W2, b2[None, :], g2[None, :], be2[None, :],
        W3, b3[None, :],
    )


# speedup vs baseline: 48.1223x; 1.0317x over previous
import functools

import jax
import jax.numpy as jnp
from jax import lax
from jax.experimental import pallas as pl
from jax.experimental.pallas import tpu as pltpu
from jax.experimental.pallas import tpu_sc as plsc

F = 26
V = 100000
E = 16
NUMERIC = 13
B = 16384
H1, H2, H3 = 256, 128, 3
EPS = 1e-5

NC = 2
NS = 16
NW = NC * NS

TASKS = F * E
TPW = TASKS // NW
NPASS = 4
QV = 25088
TAIL = V - 74880 - QV
PASSES = (
    (0, 0, 0, QV, QV),
    (QV, QV, QV, 2 * QV, QV),
    (2 * QV, 0, 2 * QV, 3 * QV, QV),
    (74880, QV, 3 * QV, V, QV + TAIL),
)


def _sc_gather_body(tab_hbm, tail_hbm, idx_hbm, out_hbm, row_v, idx_v, out_v,
                    sem0, sem1, tsem, isem, osem0, osem1):
    c = lax.axis_index("c")
    s = lax.axis_index("s")
    wid = s * NC + c
    tid0 = wid * TPW

    def rowcp(t, q):
        tid = tid0 + t
        src, slot, _, _, _ = PASSES[q]
        return pltpu.make_async_copy(
            tab_hbm.at[tid // E, tid % E, pl.ds(src, QV)],
            row_v.at[pl.ds(slot, QV)],
            sem0 if q % 2 == 0 else sem1,
        )

    def tailcp(t):
        tid = tid0 + t
        return pltpu.make_async_copy(
            tail_hbm.at[tid // E, tid % E],
            row_v.at[pl.ds(2 * QV, 128)],
            tsem,
        )

    def idxcp(t):
        tid = tid0 + t
        return pltpu.make_async_copy(
            idx_hbm.at[tid // E],
            idx_v.at[pl.ds(lax.rem(t, 2) * B, B)],
            isem,
        )

    def outcp(t, sem_):
        return pltpu.make_async_copy(
            out_v.at[pl.ds(lax.rem(t, 2) * B, B)], out_hbm.at[tid0 + t], sem_
        )

    rowcp(0, 0).start()
    idxcp(0).start()

    def task(t, carry):
        ob = lax.rem(t, 2)
        obase = ob * B
        idxcp(t).wait()

        @pl.when(t + 1 < TPW)
        def _():
            idxcp(t + 1).start()

        @pl.when((t >= 2) & (ob == 0))
        def _():
            outcp(t, osem0).wait()

        @pl.when((t >= 2) & (ob == 1))
        def _():
            outcp(t, osem1).wait()

        for q in range(NPASS):
            if q + 1 < NPASS:
                rowcp(t, q + 1).start()
                if q + 1 == NPASS - 1:
                    tailcp(t).start()
            else:
                @pl.when(t + 1 < TPW)
                def _():
                    rowcp(t + 1, 0).start()

            rowcp(t, q).wait()
            if q == NPASS - 1:
                tailcp(t).wait()

            lo, slot, mlo, mhi, ln = PASSES[q]

            @plsc.parallel_loop(0, B // 16, 1, unroll=8)
            def g16(k):
                i = obase + k * 16
                idx16 = idx_v[pl.ds(i, 16)]
                loc = jnp.clip(idx16 - lo, 0, ln - 1) + slot
                v = plsc.load_gather(row_v, [loc])
                m = (idx16 >= mlo) & (idx16 < mhi)
                pos = i + lax.iota(jnp.int32, 16)
                plsc.store_scatter(out_v, [pos], v, mask=m)

        @pl.when(ob == 0)
        def _():
            outcp(t, osem0).start()

        @pl.when(ob == 1)
        def _():
            outcp(t, osem1).start()

        return carry

    lax.fori_loop(0, TPW, task, 0)
    outcp(0, osem0).wait()
    outcp(0, osem1).wait()


@functools.cache
def _sc_gather():
    return functools.partial(
        pl.kernel,
        out_type=jax.ShapeDtypeStruct((TASKS, B), jnp.float32),
        mesh=plsc.VectorSubcoreMesh(core_axis_name="c", subcore_axis_name="s"),
        scratch_types=[
            pltpu.VMEM((2 * QV + 128,), jnp.float32),
            pltpu.VMEM((2 * B,), jnp.int32),
            pltpu.VMEM((2 * B,), jnp.float32),
            pltpu.SemaphoreType.DMA,
            pltpu.SemaphoreType.DMA,
            pltpu.SemaphoreType.DMA,
            pltpu.SemaphoreType.DMA,
            pltpu.SemaphoreType.DMA,
            pltpu.SemaphoreType.DMA,
        ],
        compiler_params=pltpu.CompilerParams(
            use_tc_tiling_on_sc=True, needs_layout_passes=False
        ),
    )(_sc_gather_body)


BT = 4096


def _mlp_body(xn_ref, embt_ref, w1n_ref, w1e_ref, b1_ref, g1_ref, be1_ref,
              w2_ref, b2_ref, g2_ref, be2_ref, w3_ref, b3_ref, out_ref):
    s = 1.0 / jnp.sqrt(1.0 + EPS)
    h = jnp.dot(xn_ref[...], w1n_ref[...], preferred_element_type=jnp.float32)
    h = h + lax.dot_general(
        embt_ref[...], w1e_ref[...], (((0,), (0,)), ((), ())),
        preferred_element_type=jnp.float32,
    )
    h = (h + b1_ref[...]) * (g1_ref[...] * s) + be1_ref[...]
    h = jnp.maximum(h, 0.0)
    h = jnp.dot(h, w2_ref[...], preferred_element_type=jnp.float32)
    h = (h + b2_ref[...]) * (g2_ref[...] * s) + be2_ref[...]
    h = jnp.maximum(h, 0.0)
    out_ref[...] = (
        lax.dot_general(w3_ref[...], h, (((0,), (1,)), ((), ())),
                        preferred_element_type=jnp.float32)
        + b3_ref[...]
    )


def _mlp(xn, embt, w1n, w1e, b1, g1, be1, w2, b2, g2, be2, w3, b3):
    full = lambda shape: pl.BlockSpec(shape, lambda i: (0,) * len(shape))
    return pl.pallas_call(
        _mlp_body,
        grid=(B // BT,),
        in_specs=[
            pl.BlockSpec((BT, NUMERIC), lambda i: (i, 0)),
            pl.BlockSpec((TASKS, BT), lambda i: (0, i)),
            full((NUMERIC, H1)),
            full((TASKS, H1)),
            full((1, H1)),
            full((1, H1)),
            full((1, H1)),
            full((H1, H2)),
            full((1, H2)),
            full((1, H2)),
            full((1, H2)),
            full((H2, H3)),
            full((H3, 1)),
        ],
        out_specs=pl.BlockSpec((H3, BT), lambda i: (0, i)),
        out_shape=jax.ShapeDtypeStruct((H3, B), jnp.float32),
    )(xn, embt, w1n, w1e, b1, g1, be1, w2, b2, g2, be2, w3, b3)


def kernel(x_numeric, x_cat, tables, W1, b1, g1, be1, W2, b2, g2, be2, W3, b3):
    tabT = jnp.transpose(tables, (0, 2, 1))
    tail = jnp.pad(tabT[:, :, V - TAIL:], ((0, 0), (0, 0), (0, 128 - TAIL)))
    idxT = jnp.clip(x_cat, 0, V - 1).T.astype(jnp.int32)
    embT = _sc_gather()(tabT, tail, idxT)
    outT = _mlp(
        x_numeric, embT, W1[:NUMERIC], W1[NUMERIC:],
        b1[None, :], g1[None, :], be1[None, :],
        W2, b2[None, :], g2[None, :], be2[None, :],
        W3, b3[:, None],
    )
    return outT.T
